# Initial kernel scaffold; baseline (speedup 1.0000x reference)
import jax
import jax.numpy as jnp
from jax.experimental import pallas as pl


def kernel(token_ids, weight):
    raise NotImplementedError("write your pallas kernel here")


# speedup vs baseline: 1.6850x; 1.6850x over previous
import functools

import jax
import jax.numpy as jnp
from jax import lax
from jax.experimental import pallas as pl
from jax.experimental.pallas import tpu as pltpu
from jax.experimental.pallas import tpu_sc as plsc

NUM_CORES = 2
NUM_SUBCORES = 16
NUM_WORKERS = NUM_CORES * NUM_SUBCORES
CHUNK = 128


@functools.lru_cache(maxsize=None)
def _build(n_idx: int, vocab: int, dim: int):
    assert n_idx % (NUM_WORKERS * CHUNK) == 0
    chunks_per_worker = n_idx // (NUM_WORKERS * CHUNK)
    rows_per_worker = chunks_per_worker * CHUNK
    mesh = plsc.VectorSubcoreMesh(
        core_axis_name="c", subcore_axis_name="s",
        num_cores=NUM_CORES, num_subcores=NUM_SUBCORES)

    @functools.partial(
        pl.kernel,
        mesh=mesh,
        out_type=jax.ShapeDtypeStruct((n_idx, dim), jnp.float32),
        scratch_types=[
            pltpu.VMEM((chunks_per_worker, CHUNK), jnp.int32),
            pltpu.VMEM((CHUNK, dim), jnp.float32),
            pltpu.SemaphoreType.DMA,
        ],
        compiler_params=pltpu.CompilerParams(use_tc_tiling_on_sc=False),
    )
    def gather_kernel(idx_hbm, table_hbm, out_hbm, idx_v, buf, gsem):
        wid = lax.axis_index("s") * NUM_CORES + lax.axis_index("c")
        row0 = wid * rows_per_worker
        pltpu.sync_copy(
            idx_hbm.at[pl.ds(wid * chunks_per_worker, chunks_per_worker)],
            idx_v)

        def body(j, carry):
            pltpu.async_copy(table_hbm.at[idx_v.at[j]], buf, gsem).wait()
            pltpu.sync_copy(buf, out_hbm.at[pl.ds(row0 + j * CHUNK, CHUNK)])
            return carry

        lax.fori_loop(0, chunks_per_worker, body, 0)

    return gather_kernel


def kernel(token_ids, weight):
    b, s = token_ids.shape
    n = b * s
    dim = weight.shape[1]
    idx = token_ids.reshape(n // CHUNK, CHUNK).astype(jnp.int32)
    fn = _build(n, weight.shape[0], dim)
    out = fn(idx, weight)
    return out.reshape(b, s, dim)


# speedup vs baseline: 1.8719x; 1.1109x over previous
import functools

import jax
import jax.numpy as jnp
from jax import lax
from jax.experimental import pallas as pl
from jax.experimental.pallas import tpu as pltpu
from jax.experimental.pallas import tpu_sc as plsc

NUM_CORES = 2
NUM_SUBCORES = 16
NUM_WORKERS = NUM_CORES * NUM_SUBCORES
CHUNK = 128
NBUF = 8


@functools.lru_cache(maxsize=None)
def _build(n_idx: int, vocab: int, dim: int):
    assert n_idx % (NUM_WORKERS * CHUNK) == 0
    chunks_per_worker = n_idx // (NUM_WORKERS * CHUNK)
    assert chunks_per_worker % NBUF == 0
    nblocks = chunks_per_worker // NBUF
    rows_per_worker = chunks_per_worker * CHUNK
    mesh = plsc.VectorSubcoreMesh(
        core_axis_name="c", subcore_axis_name="s",
        num_cores=NUM_CORES, num_subcores=NUM_SUBCORES)

    @functools.partial(
        pl.kernel,
        mesh=mesh,
        out_type=jax.ShapeDtypeStruct((n_idx, dim), jnp.float32),
        scratch_types=(
            [pltpu.VMEM((chunks_per_worker, CHUNK), jnp.int32)]
            + [pltpu.VMEM((CHUNK, dim), jnp.float32)] * NBUF
            + [pltpu.SemaphoreType.DMA] * (2 * NBUF)
        ),
        compiler_params=pltpu.CompilerParams(use_tc_tiling_on_sc=False),
    )
    def gather_kernel(idx_hbm, table_hbm, out_hbm, idx_v, *rest):
        bufs = rest[:NBUF]
        gsems = rest[NBUF:2 * NBUF]
        ssems = rest[2 * NBUF:3 * NBUF]
        wid = lax.axis_index("s") * NUM_CORES + lax.axis_index("c")
        row0 = wid * rows_per_worker
        pltpu.sync_copy(
            idx_hbm.at[pl.ds(wid * chunks_per_worker, chunks_per_worker)],
            idx_v)

        def start_gather(j, b):
            pltpu.async_copy(table_hbm.at[idx_v.at[j]], bufs[b], gsems[b])

        def wait_gather(b):
            pltpu.make_async_copy(table_hbm.at[idx_v.at[0]], bufs[b],
                                  gsems[b]).wait()

        def start_scatter(j, b):
            pltpu.async_copy(bufs[b],
                             out_hbm.at[pl.ds(row0 + j * CHUNK, CHUNK)],
                             ssems[b])

        def wait_scatter(b):
            pltpu.make_async_copy(bufs[b],
                                  out_hbm.at[pl.ds(row0, CHUNK)],
                                  ssems[b]).wait()

        for b in range(NBUF):
            start_gather(b, b)

        def block(i, carry):
            jj = i * NBUF
            for b in range(NBUF):
                wait_gather(b)
                start_scatter(jj + b, b)
            for b in range(NBUF):
                wait_scatter(b)
                start_gather(jj + NBUF + b, b)
            return carry

        lax.fori_loop(0, nblocks - 1, block, 0)

        jj = (nblocks - 1) * NBUF
        for b in range(NBUF):
            wait_gather(b)
            start_scatter(jj + b, b)
        for b in range(NBUF):
            wait_scatter(b)

    return gather_kernel


def kernel(token_ids, weight):
    b, s = token_ids.shape
    n = b * s
    dim = weight.shape[1]
    idx = token_ids.reshape(n // CHUNK, CHUNK).astype(jnp.int32)
    fn = _build(n, weight.shape[0], dim)
    out = fn(idx, weight)
    return out.reshape(b, s, dim)
